# baseline (speedup 1.0000x reference)
import functools

import jax
import jax.numpy as jnp
from jax import lax
from jax.experimental import pallas as pl
from jax.experimental.pallas import tpu as pltpu
from jax.experimental.pallas import tpu_sc as plsc

_BATCH = 16384
_FIELDS = 26
_DIM = 64
_TOTAL = _BATCH * _FIELDS


def _build():
    info = plsc.get_sparse_core_info()
    nc, ns = info.num_cores, info.num_subcores
    nw = nc * ns
    rows_per_w = _TOTAL // nw
    assert rows_per_w * nw == _TOTAL

    chunk = 1024
    n_chunks = rows_per_w // chunk
    assert n_chunks * chunk == rows_per_w
    g = 128
    n_g = chunk // g

    mesh = plsc.VectorSubcoreMesh(core_axis_name="c", subcore_axis_name="s")

    @functools.partial(
        pl.kernel,
        mesh=mesh,
        out_type=jax.ShapeDtypeStruct((_TOTAL, _DIM), jnp.float32),
        scratch_types=[
            pltpu.VMEM((chunk,), jnp.int32),
            pltpu.VMEM((chunk, _DIM), jnp.float32),
            pltpu.SemaphoreType.DMA,
        ],
        compiler_params=pltpu.CompilerParams(use_tc_tiling_on_sc=False),
    )
    def emb(idx_hbm, table_hbm, out_hbm, idx_v, rows_v, sem):
        wid = lax.axis_index("s") * nc + lax.axis_index("c")
        base = wid * rows_per_w

        def body(ci, carry):
            cbase = base + ci * chunk
            pltpu.sync_copy(idx_hbm.at[pl.ds(cbase, chunk)], idx_v)
            copies = []
            for j in range(n_g):
                copies.append(pltpu.async_copy(
                    table_hbm.at[idx_v.at[pl.ds(j * g, g)]],
                    rows_v.at[pl.ds(j * g, g)],
                    sem,
                ))
            for c in copies:
                c.wait()
            pltpu.sync_copy(rows_v, out_hbm.at[pl.ds(cbase, chunk)])
            return carry

        lax.fori_loop(0, n_chunks, body, 0)

    return emb


_emb = _build()


def kernel(input, table):
    idx_flat = input.reshape(_TOTAL)
    out = _emb(idx_flat, table)
    return out.reshape(_BATCH, _FIELDS, _DIM)


# speedup vs baseline: 1.0072x; 1.0072x over previous
import functools

import jax
import jax.numpy as jnp
from jax import lax
from jax.experimental import pallas as pl
from jax.experimental.pallas import tpu as pltpu
from jax.experimental.pallas import tpu_sc as plsc

_BATCH = 16384
_FIELDS = 26
_DIM = 64
_TOTAL = _BATCH * _FIELDS


def _build():
    info = plsc.get_sparse_core_info()
    nc, ns = info.num_cores, info.num_subcores
    nw = nc * ns
    rows_per_w = _TOTAL // nw
    assert rows_per_w * nw == _TOTAL

    nbuf = 4
    chunk = 256
    n_chunks = rows_per_w // chunk
    assert n_chunks * chunk == rows_per_w
    n_groups = n_chunks // nbuf
    assert n_groups * nbuf == n_chunks
    g = 128
    n_g = chunk // g

    mesh = plsc.VectorSubcoreMesh(core_axis_name="c", subcore_axis_name="s")

    @functools.partial(
        pl.kernel,
        mesh=mesh,
        out_type=jax.ShapeDtypeStruct((_TOTAL, _DIM), jnp.float32),
        scratch_types=[
            pltpu.VMEM((rows_per_w,), jnp.int32),
        ] + [pltpu.VMEM((chunk, _DIM), jnp.float32) for _ in range(nbuf)]
          + [pltpu.SemaphoreType.DMA for _ in range(2 * nbuf)],
        compiler_params=pltpu.CompilerParams(use_tc_tiling_on_sc=False),
    )
    def emb(idx_hbm, table_hbm, out_hbm, idx_all, *bufs_and_sems):
        rows = bufs_and_sems[:nbuf]
        sem_g = bufs_and_sems[nbuf:2 * nbuf]
        sem_s = bufs_and_sems[2 * nbuf:]
        wid = lax.axis_index("s") * nc + lax.axis_index("c")
        base = wid * rows_per_w
        pltpu.sync_copy(idx_hbm.at[pl.ds(base, rows_per_w)], idx_all)

        def group(i, carry):
            s = i * nbuf
            handles = []
            for b in range(nbuf):
                cstart = (s + b) * chunk

                @pl.when(i >= 1)
                def _():
                    pltpu.make_async_copy(
                        rows[b], out_hbm.at[pl.ds(base, chunk)], sem_s[b]
                    ).wait()

                hb = []
                for j in range(n_g):
                    hb.append(pltpu.async_copy(
                        table_hbm.at[idx_all.at[pl.ds(cstart + j * g, g)]],
                        rows[b].at[pl.ds(j * g, g)],
                        sem_g[b],
                    ))
                handles.append(hb)
            for b in range(nbuf):
                for c in handles[b]:
                    c.wait()
                pltpu.async_copy(
                    rows[b],
                    out_hbm.at[pl.ds(base + (s + b) * chunk, chunk)],
                    sem_s[b],
                )
            return carry

        lax.fori_loop(0, n_groups, group, 0)
        for b in range(nbuf):
            pltpu.make_async_copy(
                rows[b], out_hbm.at[pl.ds(base, chunk)], sem_s[b]
            ).wait()

    return emb


_emb = _build()


def kernel(input, table):
    idx_flat = input.reshape(_TOTAL)
    out = _emb(idx_flat, table)
    return out.reshape(_BATCH, _FIELDS, _DIM)
